# baseline (speedup 1.0000x reference)
import dataclasses
import functools

import jax
import jax.numpy as jnp
from jax import lax
from jax.experimental import pallas as pl
from jax.experimental.pallas import tpu as pltpu
from jax.experimental.pallas import tpu_sc as plsc

_D = 64
_S = 200
_NC = 2
_NS = 16
_NW = _NC * _NS
_BB = 8
_BPB = 128
_SR = _NW // _BB
_SPT = _S // _SR
_NB = 2


def _compiler_params():
    cp = pltpu.CompilerParams(use_tc_tiling_on_sc=False)
    if "needs_layout_passes" in pltpu.CompilerParams.__dataclass_fields__:
        cp = dataclasses.replace(cp, needs_layout_passes=False)
    return cp


def _tpe_sc(x, emb, pos):
    mesh = plsc.VectorSubcoreMesh(core_axis_name="c", subcore_axis_name="s")

    @functools.partial(
        pl.kernel,
        mesh=mesh,
        compiler_params=_compiler_params(),
        out_type=jax.ShapeDtypeStruct((_S, _D // 8, _BB, 8, _BPB), jnp.float32),
        scratch_types=[
            pltpu.VMEM((_SPT, _D), jnp.float32),
            pltpu.VMEM((_BPB, _S), jnp.int32),
            pltpu.VMEM((_SPT, _BPB), jnp.int32),
            pltpu.VMEM((_NB, _BPB, _D), jnp.float32),
            pltpu.VMEM((_NB, _D, _BPB), jnp.float32),
            pltpu.SemaphoreType.DMA((_NB,)),
            pltpu.SemaphoreType.DMA((_NB,)),
        ],
    )
    def k(emb_hbm, x_hbm, pos_hbm, out5, pos_v, xblk, idx_t, rows, stg,
          gsem, osem):
        wid = lax.axis_index("s") * _NC + lax.axis_index("c")
        bb = wid // _SR
        sr = wid % _SR
        iota = lax.iota(jnp.int32, 16)

        pltpu.sync_copy(x_hbm.at[pl.ds(bb * _BPB, _BPB)], xblk)
        pltpu.sync_copy(pos_hbm.at[pl.ds(sr * _SPT, _SPT)], pos_v)

        @pl.loop(0, _SPT)
        def _(s):
            col = jnp.full((16,), sr * _SPT + s, jnp.int32)
            for bseg in range(_BPB // 16):
                v = plsc.load_gather(xblk, [bseg * 16 + iota, col])
                idx_t.at[s].at[pl.ds(bseg * 16, 16)][...] = v

        def start_gather(s, j):
            pltpu.async_copy(emb_hbm.at[idx_t.at[s]], rows.at[j], gsem.at[j])

        for j in range(_NB):
            start_gather(j, j)

        @pl.loop(0, _SPT, step=_NB)
        def _(c):
            for j in range(_NB):
                s = c + j
                pltpu.make_async_copy(emb_hbm.at[pl.ds(0, _BPB)], rows.at[j],
                                      gsem.at[j]).wait()

                @pl.when(s >= _NB)
                def _():
                    for _tr in range(_D // 8):
                        pltpu.make_async_copy(stg.at[j].at[pl.ds(0, 8)],
                                              out5.at[0, 0, 0],
                                              osem.at[j]).wait()

                s_splat = jnp.full((16,), s, jnp.int32)

                @plsc.parallel_loop(0, _D, unroll=2)
                def _(d):
                    dcol = jnp.full((16,), d, jnp.int32)
                    pos_splat = plsc.load_gather(pos_v, [s_splat, dcol])
                    for bseg in range(_BPB // 16):
                        v = plsc.load_gather(rows.at[j],
                                             [bseg * 16 + iota, dcol])
                        stg.at[j].at[d].at[pl.ds(bseg * 16, 16)][...] = (
                            v + pos_splat)

                s_glob = sr * _SPT + s
                for tr in range(_D // 8):
                    pltpu.async_copy(stg.at[j].at[pl.ds(tr * 8, 8)],
                                     out5.at[s_glob, tr, bb], osem.at[j])

                @pl.when(s + _NB < _SPT)
                def _():
                    start_gather(s + _NB, j)

        for j in range(_NB):
            for _tr in range(_D // 8):
                pltpu.make_async_copy(stg.at[j].at[pl.ds(0, 8)],
                                      out5.at[0, 0, 0], osem.at[j]).wait()

    return k(emb, x, pos)


def kernel(x, embedding, pos_embedding):
    out5 = _tpe_sc(x.astype(jnp.int32), embedding, pos_embedding)
    return out5.transpose(2, 4, 0, 1, 3).reshape(_BB * _BPB, _S, _D)
